# Initial kernel scaffold; baseline (speedup 1.0000x reference)
import jax
import jax.numpy as jnp
from jax.experimental import pallas as pl


def kernel(x, edge_index, W1, b1, W2, b2):
    raise NotImplementedError("write your pallas kernel here")


# speedup vs baseline: 31.2023x; 31.2023x over previous
import functools

import jax
import jax.numpy as jnp
from jax import lax
from jax.experimental import pallas as pl
from jax.experimental.pallas import tpu as pltpu
from jax.experimental.pallas import tpu_sc as plsc

NC = 2
NS = 16
NW = NC * NS
CHUNK = 128


def _sc_mesh():
  return plsc.VectorSubcoreMesh(core_axis_name="c", subcore_axis_name="s")


def _copy_out(sh_ref, out_slice, s, N):
  R = 8 * ((N + 8 * NS - 1) // (8 * NS))
  full = N // R
  rem = N - full * R

  @pl.when(s < full)
  def _():
    pltpu.sync_copy(sh_ref.at[pl.ds(s * R, R)],
                    out_slice.at[pl.ds(s * R, R)])

  if rem:
    @pl.when(s == full)
    def _():
      pltpu.sync_copy(sh_ref.at[pl.ds(full * R, rem)],
                      out_slice.at[pl.ds(full * R, rem)])


def _make_deg_kernel(N, NP, C):
  rows_per_tile_zero = NP // NS

  @functools.partial(
      pl.kernel,
      out_type=jax.ShapeDtypeStruct((NC, N, 16), jnp.float32),
      mesh=_sc_mesh(),
      scratch_types=[
          pltpu.VMEM((C, CHUNK), jnp.int32),
          pltpu.VMEM((CHUNK, 16), jnp.float32),
          pltpu.VMEM_SHARED((NP, 16), jnp.float32),
      ],
  )
  def deg_kernel(dst_hbm, zeros_hbm, ones_hbm, out_hbm, dstv, ones_v, deg_sh):
    c = lax.axis_index("c")
    s = lax.axis_index("s")
    wid = c * NS + s
    pltpu.sync_copy(dst_hbm.at[wid], dstv)
    pltpu.sync_copy(ones_hbm, ones_v)
    z0 = s * rows_per_tile_zero
    pltpu.sync_copy(zeros_hbm, deg_sh.at[pl.ds(z0, rows_per_tile_zero)])
    plsc.subcore_barrier()

    def body(j, carry):
      pltpu.sync_copy(ones_v, deg_sh.at[dstv.at[j]], add=True)
      return carry

    lax.fori_loop(0, C, body, 0, unroll=False)
    plsc.subcore_barrier()
    _copy_out(deg_sh, out_hbm.at[c], s, N)

  return deg_kernel


def _make_edge_kernel(N, NP, C):
  rows_per_tile_zero = NP // NS

  @functools.partial(
      pl.kernel,
      out_type=jax.ShapeDtypeStruct((NC, N, 16), jnp.float32),
      mesh=_sc_mesh(),
      scratch_types=[
          pltpu.VMEM((C, CHUNK), jnp.int32),
          pltpu.VMEM((C, CHUNK), jnp.int32),
          pltpu.VMEM((CHUNK, 16), jnp.float32),
          pltpu.VMEM_SHARED((NP, 16), jnp.float32),
          pltpu.SemaphoreType.DMA,
      ],
      compiler_params=pltpu.CompilerParams(use_tc_tiling_on_sc=False),
  )
  def edge_kernel(src_hbm, dst_hbm, y_hbm, zeros_hbm, out_hbm,
                  srcv, dstv, rows_v, acc_sh, sem):
    c = lax.axis_index("c")
    s = lax.axis_index("s")
    wid = c * NS + s
    pltpu.sync_copy(src_hbm.at[wid], srcv)
    pltpu.sync_copy(dst_hbm.at[wid], dstv)
    z0 = s * rows_per_tile_zero
    pltpu.sync_copy(zeros_hbm, acc_sh.at[pl.ds(z0, rows_per_tile_zero)])
    plsc.subcore_barrier()

    def body(j, carry):
      pltpu.async_copy(y_hbm.at[srcv.at[j]], rows_v, sem).wait()
      pltpu.sync_copy(rows_v, acc_sh.at[dstv.at[j]], add=True)
      return carry

    lax.fori_loop(0, C, body, 0, unroll=False)
    plsc.subcore_barrier()
    _copy_out(acc_sh, out_hbm.at[c], s, N)

  return edge_kernel


def _dense1_body(x_ref, w1_ref, deg_ref, y1_ref, dv_ref):
  dv = lax.rsqrt(1.0 + deg_ref[0] + deg_ref[1])
  xw = jnp.dot(x_ref[...], w1_ref[...], preferred_element_type=jnp.float32)
  dv_ref[...] = dv
  y1_ref[...] = xw * dv


def _dense2_body(y1_ref, acc_ref, dv_ref, b1_ref, w2_ref, y2_ref):
  dv = dv_ref[...]
  h = jnp.maximum(
      dv * (acc_ref[0] + acc_ref[1] + y1_ref[...]) + b1_ref[...], 0.0)
  y2_ref[...] = jnp.dot(
      h, w2_ref[...], preferred_element_type=jnp.float32) * dv


def _dense3_body(y2_ref, acc_ref, dv_ref, b2_ref, z_ref):
  z_ref[...] = dv_ref[...] * (acc_ref[0] + acc_ref[1] + y2_ref[...]) \
      + b2_ref[...]


def kernel(x, edge_index, W1, b1, W2, b2):
  N, d_in = x.shape
  d_h = W1.shape[1]
  d_out = W2.shape[1]
  E = edge_index.shape[1]
  assert d_h == 16 and d_out == 16

  EW = NW * CHUNK
  E_pad = ((E + EW - 1) // EW) * EW
  C = E_pad // EW
  NP = ((N + 127) // 128) * 128
  pad = E_pad - E
  src = jnp.concatenate(
      [edge_index[0].astype(jnp.int32), jnp.zeros((pad,), jnp.int32)])
  dst = jnp.concatenate(
      [edge_index[1].astype(jnp.int32), jnp.full((pad,), N, jnp.int32)])
  src = src.reshape(NW, C, CHUNK)
  dst = dst.reshape(NW, C, CHUNK)

  zeros_tile = jnp.zeros((NP // NS, 16), jnp.float32)
  ones_rows = jnp.ones((CHUNK, 16), jnp.float32)

  deg_kernel = _make_deg_kernel(N, NP, C)
  edge_kernel = _make_edge_kernel(N, NP, C)

  degp = deg_kernel(dst, zeros_tile, ones_rows)

  B = 1000
  grid = (N // B,)
  y1, dv = pl.pallas_call(
      _dense1_body,
      grid=grid,
      in_specs=[
          pl.BlockSpec((B, d_in), lambda i: (i, 0)),
          pl.BlockSpec((d_in, d_h), lambda i: (0, 0)),
          pl.BlockSpec((NC, B, 16), lambda i: (0, i, 0)),
      ],
      out_specs=[
          pl.BlockSpec((B, d_h), lambda i: (i, 0)),
          pl.BlockSpec((B, 16), lambda i: (i, 0)),
      ],
      out_shape=[
          jax.ShapeDtypeStruct((N, d_h), jnp.float32),
          jax.ShapeDtypeStruct((N, 16), jnp.float32),
      ],
  )(x, W1, degp)

  acc1 = edge_kernel(src, dst, y1, zeros_tile)

  y2 = pl.pallas_call(
      _dense2_body,
      grid=grid,
      in_specs=[
          pl.BlockSpec((B, d_h), lambda i: (i, 0)),
          pl.BlockSpec((NC, B, d_h), lambda i: (0, i, 0)),
          pl.BlockSpec((B, 16), lambda i: (i, 0)),
          pl.BlockSpec((1, d_h), lambda i: (0, 0)),
          pl.BlockSpec((d_h, d_out), lambda i: (0, 0)),
      ],
      out_specs=pl.BlockSpec((B, d_out), lambda i: (i, 0)),
      out_shape=jax.ShapeDtypeStruct((N, d_out), jnp.float32),
  )(y1, acc1, dv, b1.reshape(1, d_h), W2)

  acc2 = edge_kernel(src, dst, y2, zeros_tile)

  z = pl.pallas_call(
      _dense3_body,
      grid=grid,
      in_specs=[
          pl.BlockSpec((B, d_out), lambda i: (i, 0)),
          pl.BlockSpec((NC, B, d_out), lambda i: (0, i, 0)),
          pl.BlockSpec((B, 16), lambda i: (i, 0)),
          pl.BlockSpec((1, d_out), lambda i: (0, 0)),
      ],
      out_specs=pl.BlockSpec((B, d_out), lambda i: (i, 0)),
      out_shape=jax.ShapeDtypeStruct((N, d_out), jnp.float32),
  )(y2, acc2, dv, b2.reshape(1, d_out))

  return z


# speedup vs baseline: 44.9998x; 1.4422x over previous
import functools

import jax
import jax.numpy as jnp
from jax import lax
from jax.experimental import pallas as pl
from jax.experimental.pallas import tpu as pltpu
from jax.experimental.pallas import tpu_sc as plsc

NC = 2
NS = 16
NW = NC * NS
CHUNK = 128
NBUF = 4


def _sc_mesh():
  return plsc.VectorSubcoreMesh(core_axis_name="c", subcore_axis_name="s")


def _copy_out(sh_ref, out_slice, s, N):
  R = 8 * ((N + 8 * NS - 1) // (8 * NS))
  full = N // R
  rem = N - full * R

  @pl.when(s < full)
  def _():
    pltpu.sync_copy(sh_ref.at[pl.ds(s * R, R)],
                    out_slice.at[pl.ds(s * R, R)])

  if rem:
    @pl.when(s == full)
    def _():
      pltpu.sync_copy(sh_ref.at[pl.ds(full * R, rem)],
                      out_slice.at[pl.ds(full * R, rem)])


def _make_deg_kernel(N, NP, C):
  rows_per_tile_zero = NP // NS

  @functools.partial(
      pl.kernel,
      out_type=jax.ShapeDtypeStruct((NC, N), jnp.float32),
      mesh=_sc_mesh(),
      scratch_types=[
          pltpu.VMEM((C, CHUNK), jnp.int32),
          pltpu.VMEM((CHUNK,), jnp.float32),
          pltpu.VMEM_SHARED((NP,), jnp.float32),
          pltpu.SemaphoreType.DMA,
      ],
      compiler_params=pltpu.CompilerParams(use_tc_tiling_on_sc=False),
  )
  def deg_kernel(dst_hbm, zeros_hbm, ones_hbm, out_hbm, dstv, ones_v, deg_sh,
                 sem):
    c = lax.axis_index("c")
    s = lax.axis_index("s")
    wid = c * NS + s
    pltpu.sync_copy(dst_hbm.at[wid], dstv)
    pltpu.sync_copy(ones_hbm, ones_v)
    z0 = s * rows_per_tile_zero
    pltpu.sync_copy(zeros_hbm, deg_sh.at[pl.ds(z0, rows_per_tile_zero)])
    plsc.subcore_barrier()

    def fire(j, carry):
      pltpu.async_copy(ones_v, deg_sh.at[dstv.at[j]], sem, add=True)
      return carry

    lax.fori_loop(0, C, fire, 0, unroll=False)

    def drain(j, carry):
      pltpu.make_async_copy(ones_v, deg_sh.at[dstv.at[j]], sem).wait()
      return carry

    lax.fori_loop(0, C, drain, 0, unroll=False)
    plsc.subcore_barrier()
    _copy_out(deg_sh, out_hbm.at[c], s, N)

  return deg_kernel


def _make_edge_kernel(N, NP, C):
  rows_per_tile_zero = NP // NS

  @functools.partial(
      pl.kernel,
      out_type=jax.ShapeDtypeStruct((NC, N, 16), jnp.float32),
      mesh=_sc_mesh(),
      scratch_types=[
          pltpu.VMEM((C, CHUNK), jnp.int32),
          pltpu.VMEM((C, CHUNK), jnp.int32),
          pltpu.VMEM((NBUF, CHUNK, 16), jnp.float32),
          pltpu.VMEM_SHARED((NP, 16), jnp.float32),
          pltpu.SemaphoreType.DMA((NBUF,)),
          pltpu.SemaphoreType.DMA((NBUF,)),
      ],
      compiler_params=pltpu.CompilerParams(use_tc_tiling_on_sc=False),
  )
  def edge_kernel(src_hbm, dst_hbm, y_hbm, zeros_hbm, out_hbm,
                  srcv, dstv, rows_v, acc_sh, sem_g, sem_s):
    c = lax.axis_index("c")
    s = lax.axis_index("s")
    wid = c * NS + s
    pltpu.sync_copy(src_hbm.at[wid], srcv)
    pltpu.sync_copy(dst_hbm.at[wid], dstv)
    z0 = s * rows_per_tile_zero
    pltpu.sync_copy(zeros_hbm, acc_sh.at[pl.ds(z0, rows_per_tile_zero)])
    plsc.subcore_barrier()

    def start_gather(j):
      b = lax.rem(j, NBUF)
      pltpu.async_copy(y_hbm.at[srcv.at[j]], rows_v.at[b], sem_g.at[b])

    def wait_gather(j):
      b = lax.rem(j, NBUF)
      pltpu.make_async_copy(y_hbm.at[srcv.at[j]], rows_v.at[b],
                            sem_g.at[b]).wait()

    def start_scatter(j):
      b = lax.rem(j, NBUF)
      pltpu.async_copy(rows_v.at[b], acc_sh.at[dstv.at[j]], sem_s.at[b],
                       add=True)

    def wait_scatter(j):
      b = lax.rem(j, NBUF)
      pltpu.make_async_copy(rows_v.at[b], acc_sh.at[dstv.at[j]],
                            sem_s.at[b]).wait()

    for j in range(min(NBUF - 1, C)):
      start_gather(jnp.int32(j))

    def body(j, carry):
      wait_gather(j)
      start_scatter(j)

      @pl.when(j >= 1)
      def _():
        wait_scatter(j - 1)

      @pl.when(j + NBUF - 1 < C)
      def _():
        start_gather(j + NBUF - 1)

      return carry

    lax.fori_loop(0, C, body, 0, unroll=False)
    wait_scatter(jnp.int32(C - 1))
    plsc.subcore_barrier()
    _copy_out(acc_sh, out_hbm.at[c], s, N)

  return edge_kernel


def _dense1_body(x_ref, w1_ref, deg_ref, y1_ref, dv_ref):
  dv = lax.rsqrt(1.0 + deg_ref[0] + deg_ref[1])
  xw = jnp.dot(x_ref[...], w1_ref[...], preferred_element_type=jnp.float32)
  dv_ref[...] = dv
  y1_ref[...] = xw * dv


def _dense2_body(y1_ref, acc_ref, dv_ref, b1_ref, w2_ref, y2_ref):
  dv = dv_ref[...]
  h = jnp.maximum(
      dv * (acc_ref[0] + acc_ref[1] + y1_ref[...]) + b1_ref[...], 0.0)
  y2_ref[...] = jnp.dot(
      h, w2_ref[...], preferred_element_type=jnp.float32) * dv


def _dense3_body(y2_ref, acc_ref, dv_ref, b2_ref, z_ref):
  z_ref[...] = dv_ref[...] * (acc_ref[0] + acc_ref[1] + y2_ref[...]) \
      + b2_ref[...]


def kernel(x, edge_index, W1, b1, W2, b2):
  N, d_in = x.shape
  d_h = W1.shape[1]
  d_out = W2.shape[1]
  E = edge_index.shape[1]
  assert d_h == 16 and d_out == 16

  EW = NW * CHUNK
  E_pad = ((E + EW - 1) // EW) * EW
  C = E_pad // EW
  NP = ((N + 127) // 128) * 128
  pad = E_pad - E
  src = jnp.concatenate(
      [edge_index[0].astype(jnp.int32), jnp.zeros((pad,), jnp.int32)])
  dst = jnp.concatenate(
      [edge_index[1].astype(jnp.int32), jnp.full((pad,), N, jnp.int32)])
  src = src.reshape(NW, C, CHUNK)
  dst = dst.reshape(NW, C, CHUNK)

  zeros_tile16 = jnp.zeros((NP // NS, 16), jnp.float32)
  zeros_tile1 = jnp.zeros((NP // NS,), jnp.float32)
  ones_chunk = jnp.ones((CHUNK,), jnp.float32)

  deg_kernel = _make_deg_kernel(N, NP, C)
  edge_kernel = _make_edge_kernel(N, NP, C)

  degp = deg_kernel(dst, zeros_tile1, ones_chunk)
  degp = degp.reshape(NC, N, 1)

  B = 1000
  grid = (N // B,)
  y1, dv = pl.pallas_call(
      _dense1_body,
      grid=grid,
      in_specs=[
          pl.BlockSpec((B, d_in), lambda i: (i, 0)),
          pl.BlockSpec((d_in, d_h), lambda i: (0, 0)),
          pl.BlockSpec((NC, B, 1), lambda i: (0, i, 0)),
      ],
      out_specs=[
          pl.BlockSpec((B, d_h), lambda i: (i, 0)),
          pl.BlockSpec((B, 1), lambda i: (i, 0)),
      ],
      out_shape=[
          jax.ShapeDtypeStruct((N, d_h), jnp.float32),
          jax.ShapeDtypeStruct((N, 1), jnp.float32),
      ],
  )(x, W1, degp)

  acc1 = edge_kernel(src, dst, y1, zeros_tile16)

  y2 = pl.pallas_call(
      _dense2_body,
      grid=grid,
      in_specs=[
          pl.BlockSpec((B, d_h), lambda i: (i, 0)),
          pl.BlockSpec((NC, B, d_h), lambda i: (0, i, 0)),
          pl.BlockSpec((B, 1), lambda i: (i, 0)),
          pl.BlockSpec((1, d_h), lambda i: (0, 0)),
          pl.BlockSpec((d_h, d_out), lambda i: (0, 0)),
      ],
      out_specs=pl.BlockSpec((B, d_out), lambda i: (i, 0)),
      out_shape=jax.ShapeDtypeStruct((N, d_out), jnp.float32),
  )(y1, acc1, dv, b1.reshape(1, d_h), W2)

  acc2 = edge_kernel(src, dst, y2, zeros_tile16)

  z = pl.pallas_call(
      _dense3_body,
      grid=grid,
      in_specs=[
          pl.BlockSpec((B, d_out), lambda i: (i, 0)),
          pl.BlockSpec((NC, B, d_out), lambda i: (0, i, 0)),
          pl.BlockSpec((B, 1), lambda i: (i, 0)),
          pl.BlockSpec((1, d_out), lambda i: (0, 0)),
      ],
      out_specs=pl.BlockSpec((B, d_out), lambda i: (i, 0)),
      out_shape=jax.ShapeDtypeStruct((N, d_out), jnp.float32),
  )(y2, acc2, dv, b2.reshape(1, d_out))

  return z
